# baseline (speedup 1.0000x reference)
import functools

import jax
import jax.numpy as jnp
from jax import lax
from jax.experimental import pallas as pl
from jax.experimental.pallas import tpu as pltpu
from jax.experimental.pallas import tpu_sc as plsc

N_NODES = 10000
N_EDGES = 320000
D = 128

NC = 2
NS = 16
L = 16

CHUNK = 64
NCH0 = 236
NCH1 = 77
NBUF = 4
NQUAD = NCH0 // NBUF
E0 = NS * NCH0 * CHUNK
E1 = NS * NCH1 * CHUNK
N_PAD = 10112
ROWS_PER_TILE = N_PAD // NS


def _matmul(x, W):
    def mm_kernel(x_ref, w_ref, o_ref):
        o_ref[...] = jnp.dot(x_ref[...], w_ref[...],
                             preferred_element_type=jnp.float32)

    return pl.pallas_call(
        mm_kernel,
        grid=(10,),
        in_specs=[
            pl.BlockSpec((1000, D), lambda i: (i, 0)),
            pl.BlockSpec((D, D), lambda i: (0, 0)),
        ],
        out_specs=pl.BlockSpec((1000, D), lambda i: (i, 0)),
        out_shape=jax.ShapeDtypeStruct((N_NODES, D), jnp.float32),
    )(x, W)


_SC_MESH = plsc.VectorSubcoreMesh(
    core_axis_name="c", subcore_axis_name="s", num_cores=NC, num_subcores=NS)


@functools.partial(
    pl.kernel,
    mesh=_SC_MESH,
    out_type=jax.ShapeDtypeStruct((NC, N_PAD, D), jnp.float32),
    scratch_types=(
        [pltpu.VMEM((NCH0 // 2, 2 * CHUNK), jnp.int32)]
        + [pltpu.VMEM((NBUF, CHUNK), jnp.int32)]
        + [pltpu.VMEM((NBUF, CHUNK), jnp.float32)]
        + [pltpu.VMEM((NBUF, CHUNK, D), jnp.float32)]
        + [pltpu.VMEM_SHARED((N_PAD, D), jnp.float32)]
        + [pltpu.SemaphoreType.DMA for _ in range(4 * NBUF)]
    ),
)
def _sc_aggregate(support_hbm, src0_hbm, dst0_hbm, w0_hbm,
                  src1_hbm, dst1_hbm, w1_hbm, out_hbm,
                  src_v, dst2, w2, rows3, accum, *sems):
    dsem = sems[0:NBUF]
    wsem = sems[NBUF:2 * NBUF]
    gsem = sems[2 * NBUF:3 * NBUF]
    ssem = sems[3 * NBUF:4 * NBUF]

    c = lax.axis_index("c")
    s = lax.axis_index("s")

    def zero_row(i, _):
        for cc in range(D // L):
            rows3[0, i, pl.ds(cc * L, L)] = jnp.zeros((L,), jnp.float32)
            rows3[1, i, pl.ds(cc * L, L)] = jnp.zeros((L,), jnp.float32)
        return 0
    lax.fori_loop(0, CHUNK, zero_row, 0)
    row0 = s * ROWS_PER_TILE
    for b in range(ROWS_PER_TILE // CHUNK):
        pltpu.sync_copy(rows3.at[b % 2],
                        accum.at[pl.ds(row0 + b * CHUNK, CHUNK)])
    rem = ROWS_PER_TILE % CHUNK
    if rem:
        pltpu.sync_copy(
            rows3.at[0, pl.ds(0, rem)],
            accum.at[pl.ds(row0 + (ROWS_PER_TILE // CHUNK) * CHUNK, rem)])

    base0 = s * NCH0

    @pl.when(c == 0)
    def _():
        pltpu.sync_copy(src0_hbm.at[s], src_v)
        for b in range(NBUF):
            pltpu.async_copy(dst0_hbm.at[base0 + b], dst2.at[b], dsem[b])
            pltpu.async_copy(w0_hbm.at[base0 + b], w2.at[b], wsem[b])
            pltpu.async_copy(
                support_hbm.at[src_v.at[b // 2, pl.ds((b % 2) * CHUNK, CHUNK)]],
                rows3.at[b], gsem[b])
    plsc.subcore_barrier()

    dummy_rows = support_hbm.at[pl.ds(0, CHUNK)]
    dummy_dst = dst0_hbm.at[0]
    dummy_w = w0_hbm.at[0]

    def scale(brow, wrow):
        def scale_group(g, _):
            wv = w2[wrow, pl.ds(g * L, L)]
            for j in range(L):
                wvec = jnp.full((L,), wv[j], jnp.float32)
                r = g * L + j
                for cc in range(D // L):
                    sl = pl.ds(cc * L, L)
                    rows3[brow, r, sl] = rows3[brow, r, sl] * wvec
            return 0
        lax.fori_loop(0, CHUNK // L, scale_group, 0)

    @pl.when(c == 0)
    def _():
        def quad(q, _):
            e_base = NBUF * q
            sds = []
            for b in range(NBUF):
                pltpu.make_async_copy(dummy_rows, rows3.at[b], gsem[b]).wait()
                pltpu.make_async_copy(dummy_dst, dst2.at[b], dsem[b]).wait()
                pltpu.make_async_copy(dummy_w, w2.at[b], wsem[b]).wait()
                scale(b, b)
                sds.append(pltpu.async_copy(
                    rows3.at[b], accum.at[dst2.at[b]], ssem[b], add=True))
            for b in range(NBUF):
                sds[b].wait()
                e_next = e_base + b + NBUF

                @pl.when(q < NQUAD - 1)
                def _(b=b, e_next=e_next):
                    pltpu.async_copy(
                        dst0_hbm.at[base0 + e_next], dst2.at[b], dsem[b])
                    pltpu.async_copy(
                        w0_hbm.at[base0 + e_next], w2.at[b], wsem[b])
                    pltpu.async_copy(
                        support_hbm.at[
                            src_v.at[e_next // 2,
                                     pl.ds((e_next % 2) * CHUNK, CHUNK)]],
                        rows3.at[b], gsem[b])
            return 0
        lax.fori_loop(0, NQUAD, quad, 0)

    @pl.when(c == 1)
    def _():
        def edge_chunk(i, _):
            off = (s * NCH1 + i) * CHUNK
            pltpu.sync_copy(src1_hbm.at[pl.ds(off, CHUNK)], dst2.at[1])
            pltpu.sync_copy(dst1_hbm.at[pl.ds(off, CHUNK)], dst2.at[0])
            pltpu.sync_copy(w1_hbm.at[pl.ds(off, CHUNK)], w2.at[0])
            pltpu.async_copy(support_hbm.at[dst2.at[1]], rows3.at[0],
                             gsem[0]).wait()
            scale(0, 0)
            pltpu.sync_copy(rows3.at[0], accum.at[dst2.at[0]], add=True)
            return 0
        lax.fori_loop(0, NCH1, edge_chunk, 0)
    plsc.subcore_barrier()

    pltpu.sync_copy(accum.at[pl.ds(row0, ROWS_PER_TILE)],
                    out_hbm.at[c, pl.ds(row0, ROWS_PER_TILE)])


def _add_relu(partials):
    def ar_kernel(p_ref, o_ref):
        o_ref[...] = jnp.maximum(p_ref[0] + p_ref[1], 0.0)

    return pl.pallas_call(
        ar_kernel,
        grid=(10,),
        in_specs=[pl.BlockSpec((NC, 1000, D), lambda i: (0, i, 0))],
        out_specs=pl.BlockSpec((1000, D), lambda i: (i, 0)),
        out_shape=jax.ShapeDtypeStruct((N_NODES, D), jnp.float32),
    )(partials)


def kernel(x, edge_index, edge_weight, W):
    support = _matmul(x, W)
    dst = edge_index[0].astype(jnp.int32)
    src = edge_index[1].astype(jnp.int32)
    w = edge_weight
    src0 = src[:E0].reshape(NS, NCH0 // 2, 2 * CHUNK)
    dst0 = dst[:E0].reshape(NS * NCH0, CHUNK)
    w0 = w[:E0].reshape(NS * NCH0, CHUNK)
    pad1 = E0 + E1 - N_EDGES
    src1 = jnp.pad(src[E0:], (0, pad1))
    dst1 = jnp.pad(dst[E0:], (0, pad1))
    w1 = jnp.pad(w[E0:], (0, pad1))
    partials = _sc_aggregate(support, src0, dst0, w0, src1, dst1, w1)
    return _add_relu(partials[:, :N_NODES])
